# baseline (speedup 1.0000x reference)
import functools

import jax
import jax.numpy as jnp
from jax import lax
from jax.experimental import pallas as pl
from jax.experimental.pallas import tpu as pltpu
from jax.experimental.pallas import tpu_sc as plsc

NUM_CLASSES = 100000
DIM = 128
BATCH = 16384

NC = 2
NS = 16
NW = NC * NS
B_PER_W = BATCH // NW
CHUNK = 128
N_CHUNKS = B_PER_W // CHUNK
NSETS = 3


def _sc_gather_body(labels_hbm, t1_hbm, t2_hbm, out_hbm, idx_v, buf1, buf2,
                    gsem, wsem):
    wid = lax.axis_index("s") * NC + lax.axis_index("c")
    pltpu.sync_copy(labels_hbm.at[pl.ds(wid * N_CHUNKS, N_CHUNKS)], idx_v)

    gd = [None] * N_CHUNKS
    wd = [None] * N_CHUNKS

    def issue_gather(c):
        s = c % NSETS
        idx_c = idx_v.at[c]
        gd[c] = (pltpu.async_copy(t1_hbm.at[idx_c], buf1.at[s], gsem.at[s]),
                 pltpu.async_copy(t2_hbm.at[idx_c], buf2.at[s], gsem.at[s]))

    for c in range(min(NSETS, N_CHUNKS)):
        issue_gather(c)

    for c in range(N_CHUNKS):
        s = c % NSETS
        gd[c][0].wait()
        gd[c][1].wait()
        row0 = (wid * N_CHUNKS + c) * CHUNK
        wd[c] = (pltpu.async_copy(buf1.at[s], out_hbm.at[pl.ds(row0, CHUNK), 0],
                                  wsem.at[s]),
                 pltpu.async_copy(buf2.at[s], out_hbm.at[pl.ds(row0, CHUNK), 1],
                                  wsem.at[s]))
        nxt = c + NSETS
        if nxt < N_CHUNKS:
            wd[c][0].wait()
            wd[c][1].wait()
            wd[c] = None
            issue_gather(nxt)

    for c in range(N_CHUNKS):
        if wd[c] is not None:
            wd[c][0].wait()
            wd[c][1].wait()


def _sc_gather(labels2d, table1, table2):
    mesh = plsc.VectorSubcoreMesh(
        core_axis_name="c", subcore_axis_name="s",
        num_cores=NC, num_subcores=NS)
    k = pl.kernel(
        _sc_gather_body,
        out_type=jax.ShapeDtypeStruct((BATCH, 2, DIM), jnp.float32),
        mesh=mesh,
        scratch_types=[
            pltpu.VMEM((N_CHUNKS, CHUNK), jnp.int32),
            pltpu.VMEM((NSETS, CHUNK, DIM), jnp.float32),
            pltpu.VMEM((NSETS, CHUNK, DIM), jnp.float32),
            pltpu.SemaphoreType.DMA((NSETS,)),
            pltpu.SemaphoreType.DMA((NSETS,)),
        ],
    )
    return k(labels2d, table1, table2)


def _mlp_body(e1_ref, e2_ref, w1_ref, b1_ref, w2_ref, b2_ref, out_ref):
    e1 = e1_ref[...]
    e2 = e2_ref[...]
    w1a = w1_ref[:DIM, :]
    w1b = w1_ref[DIM:, :]
    h = (jnp.dot(e1, w1a, preferred_element_type=jnp.float32)
         + jnp.dot(e2, w1b, preferred_element_type=jnp.float32)
         + b1_ref[0, :][None, :])
    h = h * jax.nn.sigmoid(h)
    g = jnp.dot(h, w2_ref[...], preferred_element_type=jnp.float32)
    out_ref[...] = g + b2_ref[0, :][None, :]


def _mlp(emb, W1, b1, W2, b2):
    bb = 2048
    grid = (BATCH // bb,)
    return pl.pallas_call(
        _mlp_body,
        grid=grid,
        in_specs=[
            pl.BlockSpec((bb, DIM), lambda i: (i, 0)),
            pl.BlockSpec((bb, DIM), lambda i: (i, 1)),
            pl.BlockSpec((2 * DIM, DIM), lambda i: (0, 0)),
            pl.BlockSpec((1, DIM), lambda i: (0, 0)),
            pl.BlockSpec((DIM, DIM), lambda i: (0, 0)),
            pl.BlockSpec((1, DIM), lambda i: (0, 0)),
        ],
        out_specs=pl.BlockSpec((bb, DIM), lambda i: (i, 0)),
        out_shape=jax.ShapeDtypeStruct((BATCH, DIM), jnp.float32),
    )(emb, emb, W1, b1, W2, b2)


def _mlp_from_stacked(embeddings, W1, b1, W2, b2):
    emb_flat = embeddings.reshape(BATCH, 2 * DIM)
    return _mlp(emb_flat, W1, b1, W2, b2)


def kernel(labels, train, table1, table2, W1, b1, W2, b2):
    labels2d = labels.astype(jnp.int32).reshape(BATCH // CHUNK, CHUNK)
    embeddings = _sc_gather(labels2d, table1, table2)
    global_embeddings = _mlp_from_stacked(embeddings, W1, b1.reshape(1, DIM),
                                          W2, b2.reshape(1, DIM))
    return (embeddings, global_embeddings)


# speedup vs baseline: 1.2469x; 1.2469x over previous
import jax
import jax.numpy as jnp
from jax import lax
from jax.experimental import pallas as pl
from jax.experimental.pallas import tpu as pltpu
from jax.experimental.pallas import tpu_sc as plsc

NUM_CLASSES = 100000
DIM = 128
BATCH = 16384

NC = 2
NS = 16
NW = NC * NS
B_PER_W = BATCH // NW
CHUNK = 128
N_CHUNKS = B_PER_W // CHUNK
NSETS = 3


def _sc_gather_body(labels_hbm, t1_hbm, t2_hbm, out_hbm, idx_v, buf1, buf2,
                    gsem, wsem):
    wid = lax.axis_index("s") * NC + lax.axis_index("c")
    pltpu.sync_copy(labels_hbm.at[pl.ds(wid * N_CHUNKS, N_CHUNKS)], idx_v)

    gd = [None] * N_CHUNKS
    wd = [None] * N_CHUNKS

    def issue_gather(c):
        s = c % NSETS
        idx_c = idx_v.at[c]
        gd[c] = (pltpu.async_copy(t1_hbm.at[idx_c], buf1.at[s], gsem.at[s]),
                 pltpu.async_copy(t2_hbm.at[idx_c], buf2.at[s], gsem.at[s]))

    for c in range(min(NSETS, N_CHUNKS)):
        issue_gather(c)

    for c in range(N_CHUNKS):
        s = c % NSETS
        gd[c][0].wait()
        gd[c][1].wait()
        row0 = (wid * N_CHUNKS + c) * CHUNK
        wd[c] = (
            pltpu.async_copy(
                buf1.at[s], out_hbm.at[pl.ds(row0, CHUNK), pl.ds(0, DIM)],
                wsem.at[s]),
            pltpu.async_copy(
                buf2.at[s], out_hbm.at[pl.ds(row0, CHUNK), pl.ds(DIM, DIM)],
                wsem.at[s]),
        )
        nxt = c + NSETS
        if nxt < N_CHUNKS:
            wd[c][0].wait()
            wd[c][1].wait()
            wd[c] = None
            issue_gather(nxt)

    for c in range(N_CHUNKS):
        if wd[c] is not None:
            wd[c][0].wait()
            wd[c][1].wait()


def _sc_gather(labels2d, table1, table2):
    mesh = plsc.VectorSubcoreMesh(
        core_axis_name="c", subcore_axis_name="s",
        num_cores=NC, num_subcores=NS)
    k = pl.kernel(
        _sc_gather_body,
        out_type=jax.ShapeDtypeStruct((BATCH, 2 * DIM), jnp.float32),
        mesh=mesh,
        scratch_types=[
            pltpu.VMEM((N_CHUNKS, CHUNK), jnp.int32),
            pltpu.VMEM((NSETS, CHUNK, DIM), jnp.float32),
            pltpu.VMEM((NSETS, CHUNK, DIM), jnp.float32),
            pltpu.SemaphoreType.DMA((NSETS,)),
            pltpu.SemaphoreType.DMA((NSETS,)),
        ],
    )
    return k(labels2d, table1, table2)


def _mlp_body(e1_ref, e2_ref, w1_ref, b1_ref, w2_ref, b2_ref,
              emb_ref, out_ref):
    e1 = e1_ref[...]
    e2 = e2_ref[...]
    emb_ref[:, 0, :] = e1
    emb_ref[:, 1, :] = e2
    w1a = w1_ref[:DIM, :]
    w1b = w1_ref[DIM:, :]
    h = (jnp.dot(e1, w1a, preferred_element_type=jnp.float32)
         + jnp.dot(e2, w1b, preferred_element_type=jnp.float32)
         + b1_ref[0, :][None, :])
    h = h * jax.nn.sigmoid(h)
    g = jnp.dot(h, w2_ref[...], preferred_element_type=jnp.float32)
    out_ref[...] = g + b2_ref[0, :][None, :]


def _mlp(cat, W1, b1, W2, b2):
    bb = 2048
    grid = (BATCH // bb,)
    return pl.pallas_call(
        _mlp_body,
        grid=grid,
        in_specs=[
            pl.BlockSpec((bb, DIM), lambda i: (i, 0)),
            pl.BlockSpec((bb, DIM), lambda i: (i, 1)),
            pl.BlockSpec((2 * DIM, DIM), lambda i: (0, 0)),
            pl.BlockSpec((1, DIM), lambda i: (0, 0)),
            pl.BlockSpec((DIM, DIM), lambda i: (0, 0)),
            pl.BlockSpec((1, DIM), lambda i: (0, 0)),
        ],
        out_specs=[
            pl.BlockSpec((bb, 2, DIM), lambda i: (i, 0, 0)),
            pl.BlockSpec((bb, DIM), lambda i: (i, 0)),
        ],
        out_shape=[
            jax.ShapeDtypeStruct((BATCH, 2, DIM), jnp.float32),
            jax.ShapeDtypeStruct((BATCH, DIM), jnp.float32),
        ],
    )(cat, cat, W1, b1, W2, b2)


def kernel(labels, train, table1, table2, W1, b1, W2, b2):
    labels2d = labels.astype(jnp.int32).reshape(BATCH // CHUNK, CHUNK)
    cat = _sc_gather(labels2d, table1, table2)
    embeddings, global_embeddings = _mlp(cat, W1, b1.reshape(1, DIM),
                                         W2, b2.reshape(1, DIM))
    return (embeddings, global_embeddings)


# speedup vs baseline: 1.2733x; 1.0212x over previous
import jax
import jax.numpy as jnp
from jax import lax
from jax.experimental import pallas as pl
from jax.experimental.pallas import tpu as pltpu
from jax.experimental.pallas import tpu_sc as plsc

NUM_CLASSES = 100000
DIM = 128
BATCH = 16384

NC = 2
NS = 16
NW = NC * NS
B_PER_W = BATCH // NW
CHUNK = 128
N_CHUNKS = B_PER_W // CHUNK
NSETS = 3


def _sc_gather_body(labels_hbm, t1_hbm, t2_hbm, cat_hbm, stk_hbm,
                    idx_v, buf1, buf2, gsem, wsem):
    wid = lax.axis_index("s") * NC + lax.axis_index("c")
    pltpu.sync_copy(labels_hbm.at[pl.ds(wid * B_PER_W, B_PER_W)], idx_v)

    gd = [None] * N_CHUNKS
    wd = [None] * N_CHUNKS

    def issue_gather(c):
        s = c % NSETS
        idx_c = idx_v.at[pl.ds(c * CHUNK, CHUNK)]
        gd[c] = (pltpu.async_copy(t1_hbm.at[idx_c], buf1.at[s], gsem.at[s]),
                 pltpu.async_copy(t2_hbm.at[idx_c], buf2.at[s], gsem.at[s]))

    for c in range(min(NSETS, N_CHUNKS)):
        issue_gather(c)

    for c in range(N_CHUNKS):
        s = c % NSETS
        gd[c][0].wait()
        gd[c][1].wait()
        row0 = (wid * N_CHUNKS + c) * CHUNK
        rows = pl.ds(row0, CHUNK)
        wd[c] = (
            pltpu.async_copy(buf1.at[s], cat_hbm.at[rows, pl.ds(0, DIM)],
                             wsem.at[s]),
            pltpu.async_copy(buf2.at[s], cat_hbm.at[rows, pl.ds(DIM, DIM)],
                             wsem.at[s]),
            pltpu.async_copy(buf1.at[s], stk_hbm.at[rows, 0], wsem.at[s]),
            pltpu.async_copy(buf2.at[s], stk_hbm.at[rows, 1], wsem.at[s]),
        )
        nxt = c + NSETS
        if nxt < N_CHUNKS:
            for d in wd[c]:
                d.wait()
            wd[c] = None
            issue_gather(nxt)

    for c in range(N_CHUNKS):
        if wd[c] is not None:
            for d in wd[c]:
                d.wait()


def _sc_gather(labels1d, table1, table2):
    mesh = plsc.VectorSubcoreMesh(
        core_axis_name="c", subcore_axis_name="s",
        num_cores=NC, num_subcores=NS)
    k = pl.kernel(
        _sc_gather_body,
        out_type=(jax.ShapeDtypeStruct((BATCH, 2 * DIM), jnp.float32),
                  jax.ShapeDtypeStruct((BATCH, 2, DIM), jnp.float32)),
        mesh=mesh,
        scratch_types=[
            pltpu.VMEM((B_PER_W,), jnp.int32),
            pltpu.VMEM((NSETS, CHUNK, DIM), jnp.float32),
            pltpu.VMEM((NSETS, CHUNK, DIM), jnp.float32),
            pltpu.SemaphoreType.DMA((NSETS,)),
            pltpu.SemaphoreType.DMA((NSETS,)),
        ],
    )
    return k(labels1d, table1, table2)


def _mlp_body(e1_ref, e2_ref, w1_ref, b1_ref, w2_ref, b2_ref, out_ref):
    e1 = e1_ref[...]
    e2 = e2_ref[...]
    w1a = w1_ref[:DIM, :]
    w1b = w1_ref[DIM:, :]
    h = (jnp.dot(e1, w1a, preferred_element_type=jnp.float32)
         + jnp.dot(e2, w1b, preferred_element_type=jnp.float32)
         + b1_ref[0, :][None, :])
    h = h * jax.nn.sigmoid(h)
    g = jnp.dot(h, w2_ref[...], preferred_element_type=jnp.float32)
    out_ref[...] = g + b2_ref[0, :][None, :]


def _mlp(cat, W1, b1, W2, b2):
    bb = 2048
    grid = (BATCH // bb,)
    return pl.pallas_call(
        _mlp_body,
        grid=grid,
        in_specs=[
            pl.BlockSpec((bb, DIM), lambda i: (i, 0)),
            pl.BlockSpec((bb, DIM), lambda i: (i, 1)),
            pl.BlockSpec((2 * DIM, DIM), lambda i: (0, 0)),
            pl.BlockSpec((1, DIM), lambda i: (0, 0)),
            pl.BlockSpec((DIM, DIM), lambda i: (0, 0)),
            pl.BlockSpec((1, DIM), lambda i: (0, 0)),
        ],
        out_specs=pl.BlockSpec((bb, DIM), lambda i: (i, 0)),
        out_shape=jax.ShapeDtypeStruct((BATCH, DIM), jnp.float32),
    )(cat, cat, W1, b1, W2, b2)


def kernel(labels, train, table1, table2, W1, b1, W2, b2):
    labels1d = labels.astype(jnp.int32)
    cat, embeddings = _sc_gather(labels1d, table1, table2)
    global_embeddings = _mlp(cat, W1, b1.reshape(1, DIM),
                             W2, b2.reshape(1, DIM))
    return (embeddings, global_embeddings)
